# Initial kernel scaffold; baseline (speedup 1.0000x reference)
import jax
import jax.numpy as jnp
from jax.experimental import pallas as pl


def kernel(x0, edge_index, batch, params):
    raise NotImplementedError("write your pallas kernel here")


# speedup vs baseline: 15.3836x; 15.3836x over previous
import functools
import math

import jax
import jax.numpy as jnp
from jax import lax
from jax.experimental import pallas as pl
from jax.experimental.pallas import tpu as pltpu
from jax.experimental.pallas import tpu_sc as plsc

N_NODES = 10000
N_EDGES = 320000
NUM_GRAPHS = 100
D_IN = 128
HID = 32
GH = 64
K1 = 25
H = 4
DH = GH // H
NUM_CLASSES = 6
SCALE = 1.0 / math.sqrt(GH)

NC, NS = 2, 16
NW = NC * NS
NPAD = 10016
SLAB = NPAD // NS
ECH = 128
EROWS = 2560
EP = EROWS * ECH
ERPW = EROWS // NW
CH = 4
RB = 1000
CB = 512
NKP = N_NODES + CB


def _sc_mesh():
    return plsc.VectorSubcoreMesh(
        core_axis_name="c", subcore_axis_name="s",
        num_cores=NC, num_subcores=NS)


_SC_PARAMS = pltpu.CompilerParams(use_tc_tiling_on_sc=False)


def _sc_degree(dstp, ones_rows, zslab):
    @functools.partial(
        pl.kernel,
        out_type=jax.ShapeDtypeStruct((NC, NPAD, 8), jnp.float32),
        mesh=_sc_mesh(),
        compiler_params=_SC_PARAMS,
        scratch_types=[
            pltpu.VMEM((ERPW, ECH), jnp.int32),
            pltpu.VMEM((ECH, 8), jnp.float32),
            pltpu.VMEM((SLAB, 8), jnp.float32),
            pltpu.VMEM_SHARED((NPAD, 8), jnp.float32),
        ],
    )
    def k(dst_hbm, ones_hbm, z_hbm, out_hbm, dst_v, ones_v, slab_v, acc):
        c = lax.axis_index("c")
        s = lax.axis_index("s")
        wid = c * NS + s
        pltpu.sync_copy(dst_hbm.at[pl.ds(wid * ERPW, ERPW)], dst_v)
        pltpu.sync_copy(ones_hbm, ones_v)
        pltpu.sync_copy(z_hbm, slab_v)
        pltpu.sync_copy(slab_v, acc.at[pl.ds(s * SLAB, SLAB)])
        plsc.subcore_barrier()

        @pl.loop(0, ERPW)
        def _(j):
            pltpu.sync_copy(ones_v, acc.at[dst_v.at[j]], add=True)

        plsc.subcore_barrier()
        pltpu.sync_copy(acc.at[pl.ds(s * SLAB, SLAB)], slab_v)
        pltpu.sync_copy(slab_v, out_hbm.at[c, pl.ds(s * SLAB, SLAB)])

    return k(dstp, ones_rows, zslab)


def _sc_aggregate(xs, srcp, dstp, zslab, d):
    @functools.partial(
        pl.kernel,
        out_type=jax.ShapeDtypeStruct((NC, NPAD, d), jnp.float32),
        mesh=_sc_mesh(),
        compiler_params=_SC_PARAMS,
        scratch_types=(
            [pltpu.VMEM((ERPW, ECH), jnp.int32)] * 2
            + [pltpu.VMEM((ECH, d), jnp.float32)] * CH
            + [pltpu.VMEM((SLAB, d), jnp.float32),
               pltpu.VMEM_SHARED((NPAD, d), jnp.float32)]
            + [pltpu.SemaphoreType.DMA] * (2 * CH)
        ),
    )
    def k(xs_hbm, src_hbm, dst_hbm, z_hbm, out_hbm,
          src_v, dst_v, r0, r1, r2, r3, slab_v, acc,
          g0, g1, g2, g3, s0, s1, s2, s3):
        rows = (r0, r1, r2, r3)
        gsem = (g0, g1, g2, g3)
        ssem = (s0, s1, s2, s3)
        c = lax.axis_index("c")
        s = lax.axis_index("s")
        wid = c * NS + s
        pltpu.sync_copy(src_hbm.at[pl.ds(wid * ERPW, ERPW)], src_v)
        pltpu.sync_copy(dst_hbm.at[pl.ds(wid * ERPW, ERPW)], dst_v)
        pltpu.sync_copy(z_hbm, slab_v)
        pltpu.sync_copy(slab_v, acc.at[pl.ds(s * SLAB, SLAB)])
        plsc.subcore_barrier()

        @pl.loop(0, ERPW, step=CH)
        def _(j):
            hg = [pltpu.async_copy(xs_hbm.at[src_v.at[j + b]], rows[b], gsem[b])
                  for b in range(CH)]
            hs = []
            for b in range(CH):
                hg[b].wait()
                hs.append(pltpu.async_copy(rows[b], acc.at[dst_v.at[j + b]],
                                           ssem[b], add=True))
            for hb in hs:
                hb.wait()

        plsc.subcore_barrier()
        pltpu.sync_copy(acc.at[pl.ds(s * SLAB, SLAB)], slab_v)
        pltpu.sync_copy(slab_v, out_hbm.at[c, pl.ds(s * SLAB, SLAB)])

    return k(xs, srcp, dstp, zslab)


def _sc_aggregate64_split(xs2, srcp, dstp, zslab):
    hw = GH // 2
    erpt = EROWS // NS

    @functools.partial(
        pl.kernel,
        out_type=jax.ShapeDtypeStruct((NC, NPAD, hw), jnp.float32),
        mesh=_sc_mesh(),
        compiler_params=_SC_PARAMS,
        scratch_types=(
            [pltpu.VMEM((erpt, ECH), jnp.int32)] * 2
            + [pltpu.VMEM((ECH, hw), jnp.float32)] * CH
            + [pltpu.VMEM((SLAB, hw), jnp.float32),
               pltpu.VMEM_SHARED((NPAD, hw), jnp.float32)]
            + [pltpu.SemaphoreType.DMA] * (2 * CH)
        ),
    )
    def k(xs_hbm, src_hbm, dst_hbm, z_hbm, out_hbm,
          src_v, dst_v, r0, r1, r2, r3, slab_v, acc,
          g0, g1, g2, g3, s0, s1, s2, s3):
        rows = (r0, r1, r2, r3)
        gsem = (g0, g1, g2, g3)
        ssem = (s0, s1, s2, s3)
        c = lax.axis_index("c")
        s = lax.axis_index("s")
        tab = xs_hbm.at[c]
        pltpu.sync_copy(src_hbm.at[pl.ds(s * erpt, erpt)], src_v)
        pltpu.sync_copy(dst_hbm.at[pl.ds(s * erpt, erpt)], dst_v)
        pltpu.sync_copy(z_hbm, slab_v)
        pltpu.sync_copy(slab_v, acc.at[pl.ds(s * SLAB, SLAB)])
        plsc.subcore_barrier()

        @pl.loop(0, erpt, step=CH)
        def _(j):
            hg = [pltpu.async_copy(tab.at[src_v.at[j + b]], rows[b], gsem[b])
                  for b in range(CH)]
            hs = []
            for b in range(CH):
                hg[b].wait()
                hs.append(pltpu.async_copy(rows[b], acc.at[dst_v.at[j + b]],
                                           ssem[b], add=True))
            for hb in hs:
                hb.wait()

        plsc.subcore_barrier()
        pltpu.sync_copy(acc.at[pl.ds(s * SLAB, SLAB)], slab_v)
        pltpu.sync_copy(slab_v, out_hbm.at[c, pl.ds(s * SLAB, SLAB)])

    return k(xs2, srcp, dstp, zslab)


def _tc_prep1(degp, x0, w1):
    def body(deg_ref, x_ref, w_ref, dinv_ref, xw_ref, xs_ref):
        deg = deg_ref[0, :, 0:1] + deg_ref[1, :, 0:1] + 1.0
        dinv = lax.rsqrt(deg)
        xw = jnp.dot(x_ref[...], w_ref[...], preferred_element_type=jnp.float32)
        dinv_ref[...] = dinv
        xw_ref[...] = xw
        xs_ref[...] = xw * dinv

    grid = N_NODES // RB
    return pl.pallas_call(
        body,
        grid=(grid,),
        in_specs=[
            pl.BlockSpec((NC, RB, 8), lambda i: (0, i, 0)),
            pl.BlockSpec((RB, D_IN), lambda i: (i, 0)),
            pl.BlockSpec((D_IN, HID), lambda i: (0, 0)),
        ],
        out_specs=[
            pl.BlockSpec((RB, 1), lambda i: (i, 0)),
            pl.BlockSpec((RB, HID), lambda i: (i, 0)),
            pl.BlockSpec((RB, HID), lambda i: (i, 0)),
        ],
        out_shape=[
            jax.ShapeDtypeStruct((N_NODES, 1), jnp.float32),
            jax.ShapeDtypeStruct((N_NODES, HID), jnp.float32),
            jax.ShapeDtypeStruct((N_NODES, HID), jnp.float32),
        ],
    )(degp, x0, w1)


def _tc_conv(aggp, xw, dinv, b, wn):
    def body(a_ref, xw_ref, dinv_ref, b_ref, w_ref, x_ref, xwn_ref, xsn_ref):
        dinv = dinv_ref[...]
        agg = a_ref[0] + a_ref[1]
        x = jax.nn.relu(dinv * agg + dinv * dinv * xw_ref[...] + b_ref[...])
        xwn = jnp.dot(x, w_ref[...], preferred_element_type=jnp.float32)
        x_ref[...] = x
        xwn_ref[...] = xwn
        xsn_ref[...] = xwn * dinv

    grid = N_NODES // RB
    return pl.pallas_call(
        body,
        grid=(grid,),
        in_specs=[
            pl.BlockSpec((NC, RB, HID), lambda i: (0, i, 0)),
            pl.BlockSpec((RB, HID), lambda i: (i, 0)),
            pl.BlockSpec((RB, 1), lambda i: (i, 0)),
            pl.BlockSpec((1, HID), lambda i: (0, 0)),
            pl.BlockSpec((HID, HID), lambda i: (0, 0)),
        ],
        out_specs=[pl.BlockSpec((RB, HID), lambda i: (i, 0))] * 3,
        out_shape=[jax.ShapeDtypeStruct((N_NODES, HID), jnp.float32)] * 3,
    )(aggp, xw, dinv, b, wn)


def _tc_gmt(aggp, xw3, dinv, b3, x1, x2, gw, gb):
    def body(a_ref, xw_ref, dinv_ref, b_ref, x1_ref, x2_ref, gw_ref, gb_ref,
             xg_ref, xgs_ref):
        dinv = dinv_ref[...]
        agg = a_ref[0] + a_ref[1]
        x3 = jax.nn.relu(dinv * agg + dinv * dinv * xw_ref[...] + b_ref[...])
        xc = jnp.concatenate([x1_ref[...], x2_ref[...], x3], axis=1)
        xg = jnp.dot(xc, gw_ref[...], preferred_element_type=jnp.float32) \
            + gb_ref[...]
        xg_ref[...] = xg
        xgs = xg * dinv
        xgs_ref[0, :, :] = xgs[:, :GH // 2]
        xgs_ref[1, :, :] = xgs[:, GH // 2:]

    grid = N_NODES // RB
    return pl.pallas_call(
        body,
        grid=(grid,),
        in_specs=[
            pl.BlockSpec((NC, RB, HID), lambda i: (0, i, 0)),
            pl.BlockSpec((RB, HID), lambda i: (i, 0)),
            pl.BlockSpec((RB, 1), lambda i: (i, 0)),
            pl.BlockSpec((1, HID), lambda i: (0, 0)),
            pl.BlockSpec((RB, HID), lambda i: (i, 0)),
            pl.BlockSpec((RB, HID), lambda i: (i, 0)),
            pl.BlockSpec((3 * HID, GH), lambda i: (0, 0)),
            pl.BlockSpec((1, GH), lambda i: (0, 0)),
        ],
        out_specs=[pl.BlockSpec((RB, GH), lambda i: (i, 0)),
                   pl.BlockSpec((NC, RB, GH // 2), lambda i: (0, i, 0))],
        out_shape=[jax.ShapeDtypeStruct((N_NODES, GH), jnp.float32),
                   jax.ShapeDtypeStruct((NC, N_NODES, GH // 2), jnp.float32)],
    )(aggp, xw3, dinv, b3, x1, x2, gw, gb)


def _tc_kv(aggp, xg, dinv, kw, kb, vw, vb):
    def body(a_ref, xg_ref, dinv_ref, kw_ref, kb_ref, vw_ref, vb_ref,
             k_ref, v_ref):
        dinv = dinv_ref[...]
        agg = jnp.concatenate([a_ref[0], a_ref[1]], axis=1)
        sxx = dinv * agg + dinv * dinv * xg_ref[...]
        k_ref[...] = jnp.dot(sxx, kw_ref[...],
                             preferred_element_type=jnp.float32) + kb_ref[...]
        v_ref[...] = jnp.dot(sxx, vw_ref[...],
                             preferred_element_type=jnp.float32) + vb_ref[...]

    grid = N_NODES // RB
    return pl.pallas_call(
        body,
        grid=(grid,),
        in_specs=[
            pl.BlockSpec((NC, RB, GH // 2), lambda i: (0, i, 0)),
            pl.BlockSpec((RB, GH), lambda i: (i, 0)),
            pl.BlockSpec((RB, 1), lambda i: (i, 0)),
            pl.BlockSpec((GH, GH), lambda i: (0, 0)),
            pl.BlockSpec((1, GH), lambda i: (0, 0)),
            pl.BlockSpec((GH, GH), lambda i: (0, 0)),
            pl.BlockSpec((1, GH), lambda i: (0, 0)),
        ],
        out_specs=[pl.BlockSpec((RB, GH), lambda i: (i, 0))] * 2,
        out_shape=[jax.ShapeDtypeStruct((N_NODES, GH), jnp.float32)] * 2,
    )(aggp, xg, dinv, kw, kb, vw, vb)


_TAIL_WNAMES = [
    'pma1_fcq_W', 'pma1_fcq_b', 'pma1_fco_W', 'pma1_fco_b',
    'sab_fcq_W', 'sab_fcq_b', 'sab_k_W', 'sab_k_b', 'sab_v_W', 'sab_v_b',
    'sab_fco_W', 'sab_fco_b',
    'pma2_fcq_W', 'pma2_fcq_b', 'pma2_k_W', 'pma2_k_b', 'pma2_v_W',
    'pma2_v_b', 'pma2_fco_W', 'pma2_fco_b',
    'gmt_lin2_W', 'gmt_lin2_b', 'lin1_W', 'lin1_b',
]


def _softmax_att(q, k, v, h):
    qh = q[:, h * DH:(h + 1) * DH]
    kh = k[:, h * DH:(h + 1) * DH]
    vh = v[:, h * DH:(h + 1) * DH]
    sc = lax.dot_general(qh * SCALE, kh, (((1,), (1,)), ((), ())),
                         preferred_element_type=jnp.float32)
    m = jnp.max(sc, axis=1, keepdims=True)
    p = jnp.exp(sc - m)
    a = p / jnp.sum(p, axis=1, keepdims=True)
    return qh + jnp.dot(a, vh, preferred_element_type=jnp.float32)


def _tc_tail(starts, knp, vnp, s1, s2, wts, l2w_p, l2b_p):
    def body(st_ref, k_ref, v_ref, s1_ref, s2_ref, *rest):
        wr = rest[:len(_TAIL_WNAMES)]
        l2w_ref, l2b_ref, out_ref = rest[len(_TAIL_WNAMES):]
        w = {n: wr[i][...] for i, n in enumerate(_TAIL_WNAMES)}
        g = pl.program_id(0)
        s0 = st_ref[g]
        cnt = st_ref[g + 1] - s0

        q1 = jnp.dot(s1_ref[...], w['pma1_fcq_W'],
                     preferred_element_type=jnp.float32) + w['pma1_fcq_b']

        heads = []
        for h in range(H):
            qh = q1[:, h * DH:(h + 1) * DH]
            qs = qh * SCALE

            def chunk(j, carry, qs=qs, h=h):
                m, l, acc = carry
                base = s0 + j * CB
                kb = k_ref[pl.ds(base, CB), h * DH:(h + 1) * DH]
                vb = v_ref[pl.ds(base, CB), h * DH:(h + 1) * DH]
                sc = lax.dot_general(qs, kb, (((1,), (1,)), ((), ())),
                                     preferred_element_type=jnp.float32)
                valid = (j * CB + lax.broadcasted_iota(jnp.int32, (1, CB), 1)
                         ) < cnt
                sc = jnp.where(valid, sc, -1e30)
                m_new = jnp.maximum(m, jnp.max(sc, axis=1, keepdims=True))
                alpha = jnp.exp(m - m_new)
                p = jnp.where(valid, jnp.exp(sc - m_new), 0.0)
                l = l * alpha + jnp.sum(p, axis=1, keepdims=True)
                acc = acc * alpha + jnp.dot(p, vb,
                                            preferred_element_type=jnp.float32)
                return m_new, l, acc

            init = (jnp.full((K1, 1), -1e30, jnp.float32),
                    jnp.zeros((K1, 1), jnp.float32),
                    jnp.zeros((K1, DH), jnp.float32))
            carry = chunk(0, init)
            nch = (cnt + CB - 1) // CB
            m, l, acc = lax.fori_loop(1, nch, chunk, carry)
            lsafe = jnp.where(l > 0.0, l, 1.0)
            heads.append(qh + acc / lsafe)
        bx = jnp.concatenate(heads, axis=1)
        bx = bx + jax.nn.relu(
            jnp.dot(bx, w['pma1_fco_W'], preferred_element_type=jnp.float32)
            + w['pma1_fco_b'])

        qs_ = jnp.dot(bx, w['sab_fcq_W'],
                      preferred_element_type=jnp.float32) + w['sab_fcq_b']
        ks_ = jnp.dot(bx, w['sab_k_W'],
                      preferred_element_type=jnp.float32) + w['sab_k_b']
        vs_ = jnp.dot(bx, w['sab_v_W'],
                      preferred_element_type=jnp.float32) + w['sab_v_b']
        bx = jnp.concatenate([_softmax_att(qs_, ks_, vs_, h)
                              for h in range(H)], axis=1)
        bx = bx + jax.nn.relu(
            jnp.dot(bx, w['sab_fco_W'], preferred_element_type=jnp.float32)
            + w['sab_fco_b'])

        q2 = jnp.dot(s2_ref[...], w['pma2_fcq_W'],
                     preferred_element_type=jnp.float32) + w['pma2_fcq_b']
        k2 = jnp.dot(bx, w['pma2_k_W'],
                     preferred_element_type=jnp.float32) + w['pma2_k_b']
        v2 = jnp.dot(bx, w['pma2_v_W'],
                     preferred_element_type=jnp.float32) + w['pma2_v_b']
        o = jnp.concatenate([_softmax_att(q2, k2, v2, h)
                             for h in range(H)], axis=1)
        o = o + jax.nn.relu(
            jnp.dot(o, w['pma2_fco_W'], preferred_element_type=jnp.float32)
            + w['pma2_fco_b'])

        gz = jnp.dot(o, w['gmt_lin2_W'],
                     preferred_element_type=jnp.float32) + w['gmt_lin2_b']
        gz = jax.nn.relu(jnp.dot(gz, w['lin1_W'],
                                 preferred_element_type=jnp.float32)
                         + w['lin1_b'])
        logits = jnp.dot(gz, l2w_ref[...],
                         preferred_element_type=jnp.float32) + l2b_ref[...]
        lane = lax.broadcasted_iota(jnp.int32, (1, 128), 1)
        logits = jnp.where(lane < NUM_CLASSES, logits, -1e30)
        mx = jnp.max(logits, axis=1, keepdims=True)
        lse = jnp.log(jnp.sum(jnp.exp(logits - mx), axis=1, keepdims=True)) \
            + mx
        out_ref[pl.ds(g, 1), :] = logits - lse

    full = lambda shape: pl.BlockSpec(shape, lambda i, st: tuple(0 for _ in shape))
    wspecs = [full(x.shape) for x in wts]
    grid_spec = pltpu.PrefetchScalarGridSpec(
        num_scalar_prefetch=1,
        grid=(NUM_GRAPHS,),
        in_specs=[
            full((NKP, GH)), full((NKP, GH)), full((K1, GH)), full((1, GH)),
            *wspecs, full((16, 128)), full((1, 128)),
        ],
        out_specs=pl.BlockSpec((NUM_GRAPHS, 128), lambda i, st: (0, 0)),
    )
    return pl.pallas_call(
        body,
        grid_spec=grid_spec,
        out_shape=jax.ShapeDtypeStruct((NUM_GRAPHS, 128), jnp.float32),
    )(starts, knp, vnp, s1, s2, *wts, l2w_p, l2b_p)


def kernel(x0, edge_index, batch, params):
    p = params
    f32 = jnp.float32
    src = edge_index[0]
    dst = edge_index[1]
    pad = EP - N_EDGES
    srcp = jnp.concatenate(
        [src, jnp.zeros((pad,), jnp.int32)]).reshape(EROWS, ECH)
    dstp = jnp.concatenate(
        [dst, jnp.full((pad,), N_NODES, jnp.int32)]).reshape(EROWS, ECH)
    starts = jnp.searchsorted(
        batch, jnp.arange(NUM_GRAPHS + 1, dtype=jnp.int32)).astype(jnp.int32)

    ones8 = jnp.ones((ECH, 8), f32)
    z8 = jnp.zeros((SLAB, 8), f32)
    z32 = jnp.zeros((SLAB, HID), f32)

    degp = _sc_degree(dstp, ones8, z8)
    dinv, xw1, xs1 = _tc_prep1(degp, x0, p['conv1_W'])
    a1 = _sc_aggregate(xs1, srcp, dstp, z32, HID)
    x1, xw2, xs2 = _tc_conv(a1, xw1, dinv, p['conv1_b'].reshape(1, HID),
                            p['conv2_W'])
    a2 = _sc_aggregate(xs2, srcp, dstp, z32, HID)
    x2, xw3, xs3 = _tc_conv(a2, xw2, dinv, p['conv2_b'].reshape(1, HID),
                            p['conv3_W'])
    a3 = _sc_aggregate(xs3, srcp, dstp, z32, HID)
    xg, xgs = _tc_gmt(a3, xw3, dinv, p['conv3_b'].reshape(1, HID), x1, x2,
                      p['gmt_lin1_W'], p['gmt_lin1_b'].reshape(1, GH))
    ag = _sc_aggregate64_split(xgs, srcp, dstp, z32)
    kn, vn = _tc_kv(ag, xg, dinv,
                    p['pma1_k_W'], p['pma1_k_b'].reshape(1, GH),
                    p['pma1_v_W'], p['pma1_v_b'].reshape(1, GH))
    knp = jnp.pad(kn, ((0, CB), (0, 0)))
    vnp = jnp.pad(vn, ((0, CB), (0, 0)))

    wts = []
    for n in _TAIL_WNAMES:
        a = p[n]
        wts.append(a.reshape(1, a.shape[0]) if a.ndim == 1 else a)
    l2w_p = jnp.pad(p['lin2_W'], ((0, 0), (0, 128 - NUM_CLASSES)))
    l2b_p = jnp.pad(p['lin2_b'], ((0, 128 - NUM_CLASSES))).reshape(1, 128)

    out = _tc_tail(starts, knp, vnp, p['pma1_S'].reshape(K1, GH),
                   p['pma2_S'].reshape(1, GH), wts, l2w_p, l2b_p)
    return out[:, :NUM_CLASSES]
